# Initial kernel scaffold; baseline (speedup 1.0000x reference)
import jax
import jax.numpy as jnp
from jax.experimental import pallas as pl


def kernel(inputs, targets, cams, proxy, pids, cids):
    raise NotImplementedError("write your pallas kernel here")


# speedup vs baseline: 4.3816x; 4.3816x over previous
import functools

import jax
import jax.numpy as jnp
from jax import lax
from jax.experimental import pallas as pl
from jax.experimental.pallas import tpu as pltpu
from jax.experimental.pallas import tpu_sc as plsc

NUM_FEATURES = 128
NUM_SAMPLES = 100000
N_PAD = 102400
NUM_HARDS = 50
TEMP = 0.07
B = 1024
GROUP = 16
N_GROUPS = N_PAD // GROUP
COLS_PER_BLK = 2048
GRPS_PER_BLK = COLS_PER_BLK // GROUP
N_CBLK = N_PAD // COLS_PER_BLK
ROWS1 = 256
ROWS2 = 64
ROWS3 = 256
IDX_W = 64
NEG_BIG = -9999999.0
NINF = -1e30


def _pass1_body(x_ref, p_ref, tgt_ref, cam_ref, pid_ref, cid_ref,
                sm_ref, gm_ref, acc_ref):
    j = pl.program_id(1)
    x = x_ref[...]
    nrm = jnp.sqrt(jnp.sum(x * x, axis=1, keepdims=True))
    xn = x / jnp.maximum(nrm, 1e-12)
    pb = p_ref[...]
    sims = lax.dot_general(xn, pb, (((1,), (1,)), ((), ())),
                           preferred_element_type=jnp.float32) / TEMP
    t = tgt_ref[:, 0:1]
    cm = cam_ref[:, 0:1]
    p = pid_ref[0]
    cd = cid_ref[0]
    neg = (t != p) & (p >= 0.0)
    pos = (t == p) & (cm != cd)
    sm = jnp.where(neg, sims, sims + NEG_BIG)
    sm_ref[...] = sm
    gm_ref[...] = jnp.max(sm.reshape(ROWS1, GRPS_PER_BLK, GROUP), axis=2)

    bm = jnp.max(jnp.where(pos, sims, NINF), axis=1, keepdims=True)
    e = jnp.where(pos, jnp.exp(sims - bm), 0.0)
    bsum = jnp.sum(e, axis=1, keepdims=True)
    bsp = jnp.sum(jnp.where(pos, sims, 0.0), axis=1, keepdims=True)
    bcnt = jnp.sum(jnp.where(pos, 1.0, 0.0), axis=1, keepdims=True)

    @pl.when(j == 0)
    def _():
        lanes = lax.broadcasted_iota(jnp.int32, (ROWS1, 8), 1)
        acc_ref[...] = jnp.where(lanes == 0, NINF, 0.0)

    mo = acc_ref[:, 0:1]
    so = acc_ref[:, 1:2]
    mn = jnp.maximum(mo, bm)
    sn = so * jnp.exp(mo - mn) + bsum * jnp.exp(bm - mn)
    acc_ref[:, 0:1] = mn
    acc_ref[:, 1:2] = sn
    acc_ref[:, 2:3] = acc_ref[:, 2:3] + bsp
    acc_ref[:, 3:4] = acc_ref[:, 3:4] + bcnt


def _pass2_body(gm_ref, idx_ref, v_ref):
    i0 = pl.program_id(0)
    v_ref[...] = gm_ref[...]
    colio = lax.broadcasted_iota(jnp.int32, (ROWS2, N_GROUPS), 1)
    slot = lax.broadcasted_iota(jnp.int32, (ROWS2, IDX_W), 1)
    rbase = (lax.broadcasted_iota(jnp.int32, (ROWS2, 1), 0)
             + i0 * ROWS2) * N_GROUPS

    def it(i, acc):
        v = v_ref[...]
        mi = jnp.max(v, axis=1, keepdims=True)
        am = jnp.max(jnp.where(v == mi, colio, -1), axis=1, keepdims=True)
        v_ref[...] = jnp.where(colio == am, -3.0e38, v)
        gidx = jnp.broadcast_to(rbase + am, (ROWS2, IDX_W))
        return jnp.where(slot == i, gidx, acc)

    acc = lax.fori_loop(0, NUM_HARDS,
                        it, jnp.zeros((ROWS2, IDX_W), jnp.int32))
    idx_ref[...] = acc


def _sc_gather_body(idx_hbm, table_hbm, out_hbm, idx_v, rows_v, sem):
    c = lax.axis_index("c")
    s = lax.axis_index("s")
    wid = s * 2 + c
    pltpu.sync_copy(idx_hbm.at[pl.ds(wid * 16, 16)], idx_v)
    cps = []
    for k in range(16):
        cp = pltpu.async_copy(table_hbm.at[idx_v.at[k]],
                              rows_v.at[pl.ds(k * 128, 128)], sem)
        cps.append(cp)
    for cp in cps:
        cp.wait()
    pltpu.sync_copy(rows_v, out_hbm.at[pl.ds(wid * 2048, 2048)])


def _pass3_body(cand_ref, acc_ref, out_ref, v_ref):
    i0 = pl.program_id(0)
    ncand = NUM_HARDS * GROUP
    colio = lax.broadcasted_iota(jnp.int32, (ROWS3, IDX_W * GROUP), 1)
    cands = jnp.where(colio < ncand, cand_ref[...], NINF)
    v_ref[...] = cands
    m_top = jnp.max(cands, axis=1, keepdims=True)

    def it(i, stot):
        v = v_ref[...]
        mi = jnp.max(v, axis=1, keepdims=True)
        am = jnp.max(jnp.where(v == mi, colio, -1), axis=1, keepdims=True)
        v_ref[...] = jnp.where(colio == am, -3.0e38, v)
        return stot + jnp.exp(mi - m_top)

    sneg = lax.fori_loop(0, NUM_HARDS, it, jnp.zeros((ROWS3, 1), jnp.float32))

    m_pos = acc_ref[:, 0:1]
    s_pos = acc_ref[:, 1:2]
    sum_pos = acc_ref[:, 2:3]
    cnt = acc_ref[:, 3:4]
    m = jnp.maximum(m_pos, m_top)
    lse = m + jnp.log(s_pos * jnp.exp(m_pos - m) + sneg * jnp.exp(m_top - m))
    mean_pos = sum_pos / jnp.maximum(cnt, 1.0)
    per_row = jnp.where(cnt > 0, lse - mean_pos, 0.0)
    partial = jnp.sum(per_row) * (1.0 / B)

    @pl.when(i0 == 0)
    def _():
        out_ref[...] = jnp.zeros((8, 128), jnp.float32)

    out_ref[...] = out_ref[...] + lax.broadcast_in_dim(partial, (8, 128), ())


def _run_pass1(xn, proxy_p, tgt, cam, pid3, cid3):
    return pl.pallas_call(
        _pass1_body,
        grid=(B // ROWS1, N_CBLK),
        in_specs=[
            pl.BlockSpec((ROWS1, NUM_FEATURES), lambda i, j: (i, 0)),
            pl.BlockSpec((COLS_PER_BLK, NUM_FEATURES), lambda i, j: (j, 0)),
            pl.BlockSpec((ROWS1, 8), lambda i, j: (i, 0)),
            pl.BlockSpec((ROWS1, 8), lambda i, j: (i, 0)),
            pl.BlockSpec((1, 1, COLS_PER_BLK), lambda i, j: (j, 0, 0)),
            pl.BlockSpec((1, 1, COLS_PER_BLK), lambda i, j: (j, 0, 0)),
        ],
        out_specs=[
            pl.BlockSpec((ROWS1, COLS_PER_BLK), lambda i, j: (i, j)),
            pl.BlockSpec((ROWS1, GRPS_PER_BLK), lambda i, j: (i, j)),
            pl.BlockSpec((ROWS1, 8), lambda i, j: (i, 0)),
        ],
        out_shape=[
            jax.ShapeDtypeStruct((B, N_PAD), jnp.float32),
            jax.ShapeDtypeStruct((B, N_GROUPS), jnp.float32),
            jax.ShapeDtypeStruct((B, 8), jnp.float32),
        ],
    )(xn, proxy_p, tgt, cam, pid3, cid3)


def _run_pass2(gm):
    return pl.pallas_call(
        _pass2_body,
        grid=(B // ROWS2,),
        in_specs=[pl.BlockSpec((ROWS2, N_GROUPS), lambda i: (i, 0))],
        out_specs=pl.BlockSpec((ROWS2, IDX_W), lambda i: (i, 0)),
        out_shape=jax.ShapeDtypeStruct((B, IDX_W), jnp.int32),
        scratch_shapes=[pltpu.VMEM((ROWS2, N_GROUPS), jnp.float32)],
    )(gm)


@functools.partial(
    pl.kernel,
    out_type=jax.ShapeDtypeStruct((B * IDX_W, GROUP), jnp.float32),
    mesh=plsc.VectorSubcoreMesh(core_axis_name="c", subcore_axis_name="s"),
    scratch_types=[
        pltpu.VMEM((16, 128), jnp.int32),
        pltpu.VMEM((2048, GROUP), jnp.float32),
        pltpu.SemaphoreType.DMA,
    ],
    compiler_params=pltpu.CompilerParams(use_tc_tiling_on_sc=False),
)
def _sc_gather(idx_hbm, table_hbm, out_hbm, idx_v, rows_v, sem):
    _sc_gather_body(idx_hbm, table_hbm, out_hbm, idx_v, rows_v, sem)


def _run_pass3(cand, acc):
    return pl.pallas_call(
        _pass3_body,
        grid=(B // ROWS3,),
        in_specs=[
            pl.BlockSpec((ROWS3, IDX_W * GROUP), lambda i: (i, 0)),
            pl.BlockSpec((ROWS3, 8), lambda i: (i, 0)),
        ],
        out_specs=pl.BlockSpec((8, 128), lambda i: (0, 0)),
        out_shape=jax.ShapeDtypeStruct((8, 128), jnp.float32),
        scratch_shapes=[pltpu.VMEM((ROWS3, IDX_W * GROUP), jnp.float32)],
    )(cand, acc)


def kernel(inputs, targets, cams, proxy, pids, cids):
    f32 = jnp.float32
    proxy_p = jnp.pad(proxy, ((0, N_PAD - NUM_SAMPLES), (0, 0)))
    pad_i = jnp.full((N_PAD - NUM_SAMPLES,), -1, pids.dtype)
    pid3 = jnp.concatenate([pids, pad_i]).astype(f32).reshape(N_CBLK, 1,
                                                              COLS_PER_BLK)
    cid3 = jnp.concatenate([cids, pad_i]).astype(f32).reshape(N_CBLK, 1,
                                                              COLS_PER_BLK)
    tgt = jnp.broadcast_to(targets.astype(f32)[:, None], (B, 8))
    cam = jnp.broadcast_to(cams.astype(f32)[:, None], (B, 8))

    sm, gm, acc = _run_pass1(inputs.astype(f32), proxy_p, tgt, cam, pid3, cid3)
    idx = _run_pass2(gm)
    table = sm.reshape(B * N_GROUPS, GROUP)
    cand = _sc_gather(idx.reshape(B * IDX_W // 128, 128), table)
    loss = _run_pass3(cand.reshape(B, IDX_W * GROUP), acc)
    return loss[0, 0]


# speedup vs baseline: 6.5104x; 1.4858x over previous
import functools

import jax
import jax.numpy as jnp
from jax import lax
from jax.experimental import pallas as pl
from jax.experimental.pallas import tpu as pltpu
from jax.experimental.pallas import tpu_sc as plsc

NUM_FEATURES = 128
NUM_SAMPLES = 100000
N_PAD = 102400
NUM_HARDS = 50
TEMP = 0.07
B = 1024
ROWW = 128
N_COARSE = N_PAD // ROWW
COLS_PER_BLK = 2048
CRS_PER_BLK = COLS_PER_BLK // ROWW
N_CBLK = N_PAD // COLS_PER_BLK
ROWS1 = 256
ROWS2 = 256
ROWS3 = 256
IDX_W = 64
G_PER_TILE = B * IDX_W // 32
N_CHUNK = G_PER_TILE // 128
NEG_BIG = -9999999.0
NINF = -1e30


def _pass1_body(x_ref, p_ref, tgt_ref, cam_ref, pid_ref, cid_ref,
                sm_ref, gm_ref, acc_ref):
    j = pl.program_id(1)
    x = x_ref[...]
    nrm = jnp.sqrt(jnp.sum(x * x, axis=1, keepdims=True))
    xn = x / jnp.maximum(nrm, 1e-12)
    pb = p_ref[...]
    sims = lax.dot_general(xn, pb, (((1,), (1,)), ((), ())),
                           preferred_element_type=jnp.float32) / TEMP
    t = tgt_ref[:, 0:1]
    cm = cam_ref[:, 0:1]
    p = pid_ref[0]
    cd = cid_ref[0]
    neg = (t != p) & (p >= 0.0)
    pos = (t == p) & (cm != cd)
    sm = jnp.where(neg, sims, sims + NEG_BIG)
    sm3 = sm.reshape(ROWS1, CRS_PER_BLK, ROWW)
    sm_ref[...] = sm3
    gm_ref[...] = jnp.max(sm3, axis=2).reshape(1, ROWS1, CRS_PER_BLK)

    bm = jnp.max(jnp.where(pos, sims, NINF), axis=1, keepdims=True)
    e = jnp.where(pos, jnp.exp(sims - bm), 0.0)
    bsum = jnp.sum(e, axis=1, keepdims=True)
    bsp = jnp.sum(jnp.where(pos, sims, 0.0), axis=1, keepdims=True)
    bcnt = jnp.sum(jnp.where(pos, 1.0, 0.0), axis=1, keepdims=True)

    @pl.when(j == 0)
    def _():
        lanes = lax.broadcasted_iota(jnp.int32, (ROWS1, 8), 1)
        acc_ref[...] = jnp.where(lanes == 0, NINF, 0.0)

    mo = acc_ref[:, 0:1]
    so = acc_ref[:, 1:2]
    mn = jnp.maximum(mo, bm)
    sn = so * jnp.exp(mo - mn) + bsum * jnp.exp(bm - mn)
    acc_ref[:, 0:1] = mn
    acc_ref[:, 1:2] = sn
    acc_ref[:, 2:3] = acc_ref[:, 2:3] + bsp
    acc_ref[:, 3:4] = acc_ref[:, 3:4] + bcnt


def _pass2_body(gm_ref, idx_ref, v_ref):
    i0 = pl.program_id(0)
    v_ref[...] = gm_ref[...]
    colio = lax.broadcasted_iota(jnp.int32, (ROWS2, N_COARSE), 1)
    slot = lax.broadcasted_iota(jnp.int32, (ROWS2, IDX_W), 1)
    rbase = (lax.broadcasted_iota(jnp.int32, (ROWS2, 1), 0)
             + i0 * ROWS2) * N_COARSE

    def it(i, acc):
        v = v_ref[...]
        mi = jnp.max(v, axis=1, keepdims=True)
        am = jnp.max(jnp.where(v == mi, colio, -1), axis=1, keepdims=True)
        v_ref[...] = jnp.where(colio == am, -3.0e38, v)
        gidx = jnp.broadcast_to(rbase + am, (ROWS2, IDX_W))
        return jnp.where(slot == i, gidx, acc)

    acc = lax.fori_loop(0, NUM_HARDS,
                        it, jnp.zeros((ROWS2, IDX_W), jnp.int32))
    idx_ref[...] = acc


def _sc_gather_body(idx_hbm, table_hbm, out_hbm, idx_v, rows_v, sem):
    c = lax.axis_index("c")
    s = lax.axis_index("s")
    wid = s * 2 + c
    pltpu.sync_copy(idx_hbm.at[pl.ds(wid * N_CHUNK, N_CHUNK)], idx_v)
    for k in range(N_CHUNK):
        pltpu.async_copy(table_hbm.at[idx_v.at[k]], rows_v, sem).wait()
        pltpu.sync_copy(
            rows_v, out_hbm.at[pl.ds(wid * G_PER_TILE + k * 128, 128)])


def _pass3_body(cand_ref, acc_ref, out_ref, v_ref):
    i0 = pl.program_id(0)
    ncand = NUM_HARDS * ROWW
    colio = lax.broadcasted_iota(jnp.int32, (ROWS3, IDX_W * ROWW), 1)
    cands = jnp.where(colio < ncand, cand_ref[...], NINF)
    v_ref[...] = cands
    m_top = jnp.max(cands, axis=1, keepdims=True)

    def it(i, stot):
        v = v_ref[...]
        mi = jnp.max(v, axis=1, keepdims=True)
        am = jnp.max(jnp.where(v == mi, colio, -1), axis=1, keepdims=True)
        v_ref[...] = jnp.where(colio == am, -3.0e38, v)
        return stot + jnp.exp(mi - m_top)

    sneg = lax.fori_loop(0, NUM_HARDS, it, jnp.zeros((ROWS3, 1), jnp.float32))

    m_pos = acc_ref[:, 0:1]
    s_pos = acc_ref[:, 1:2]
    sum_pos = acc_ref[:, 2:3]
    cnt = acc_ref[:, 3:4]
    m = jnp.maximum(m_pos, m_top)
    lse = m + jnp.log(s_pos * jnp.exp(m_pos - m) + sneg * jnp.exp(m_top - m))
    mean_pos = sum_pos / jnp.maximum(cnt, 1.0)
    per_row = jnp.where(cnt > 0, lse - mean_pos, 0.0)
    partial = jnp.sum(per_row) * (1.0 / B)

    @pl.when(i0 == 0)
    def _():
        out_ref[...] = jnp.zeros((8, 128), jnp.float32)

    out_ref[...] = out_ref[...] + lax.broadcast_in_dim(partial, (8, 128), ())


def _run_pass1(xn, proxy_p, tgt, cam, pid3, cid3):
    return pl.pallas_call(
        _pass1_body,
        grid=(B // ROWS1, N_CBLK),
        in_specs=[
            pl.BlockSpec((ROWS1, NUM_FEATURES), lambda i, j: (i, 0)),
            pl.BlockSpec((COLS_PER_BLK, NUM_FEATURES), lambda i, j: (j, 0)),
            pl.BlockSpec((ROWS1, 8), lambda i, j: (i, 0)),
            pl.BlockSpec((ROWS1, 8), lambda i, j: (i, 0)),
            pl.BlockSpec((1, 1, COLS_PER_BLK), lambda i, j: (j, 0, 0)),
            pl.BlockSpec((1, 1, COLS_PER_BLK), lambda i, j: (j, 0, 0)),
        ],
        out_specs=[
            pl.BlockSpec((ROWS1, CRS_PER_BLK, ROWW), lambda i, j: (i, j, 0)),
            pl.BlockSpec((1, ROWS1, CRS_PER_BLK), lambda i, j: (j, i, 0)),
            pl.BlockSpec((ROWS1, 8), lambda i, j: (i, 0)),
        ],
        out_shape=[
            jax.ShapeDtypeStruct((B, N_COARSE, ROWW), jnp.float32),
            jax.ShapeDtypeStruct((N_CBLK, B, CRS_PER_BLK), jnp.float32),
            jax.ShapeDtypeStruct((B, 8), jnp.float32),
        ],
    )(xn, proxy_p, tgt, cam, pid3, cid3)


def _run_pass2(gm):
    return pl.pallas_call(
        _pass2_body,
        grid=(B // ROWS2,),
        in_specs=[pl.BlockSpec((ROWS2, N_COARSE), lambda i: (i, 0))],
        out_specs=pl.BlockSpec((ROWS2, IDX_W), lambda i: (i, 0)),
        out_shape=jax.ShapeDtypeStruct((B, IDX_W), jnp.int32),
        scratch_shapes=[pltpu.VMEM((ROWS2, N_COARSE), jnp.float32)],
    )(gm)


@functools.partial(
    pl.kernel,
    out_type=jax.ShapeDtypeStruct((B * IDX_W, ROWW), jnp.float32),
    mesh=plsc.VectorSubcoreMesh(core_axis_name="c", subcore_axis_name="s"),
    scratch_types=[
        pltpu.VMEM((N_CHUNK, 128), jnp.int32),
        pltpu.VMEM((128, ROWW), jnp.float32),
        pltpu.SemaphoreType.DMA,
    ],
)
def _sc_gather(idx_hbm, table_hbm, out_hbm, idx_v, rows_v, sem):
    _sc_gather_body(idx_hbm, table_hbm, out_hbm, idx_v, rows_v, sem)


def _run_pass3(cand, acc):
    return pl.pallas_call(
        _pass3_body,
        grid=(B // ROWS3,),
        in_specs=[
            pl.BlockSpec((ROWS3, IDX_W * ROWW), lambda i: (i, 0)),
            pl.BlockSpec((ROWS3, 8), lambda i: (i, 0)),
        ],
        out_specs=pl.BlockSpec((8, 128), lambda i: (0, 0)),
        out_shape=jax.ShapeDtypeStruct((8, 128), jnp.float32),
        scratch_shapes=[pltpu.VMEM((ROWS3, IDX_W * ROWW), jnp.float32)],
    )(cand, acc)


def kernel(inputs, targets, cams, proxy, pids, cids):
    f32 = jnp.float32
    proxy_p = jnp.pad(proxy, ((0, N_PAD - NUM_SAMPLES), (0, 0)))
    pad_i = jnp.full((N_PAD - NUM_SAMPLES,), -1, pids.dtype)
    pid3 = jnp.concatenate([pids, pad_i]).astype(f32).reshape(N_CBLK, 1,
                                                              COLS_PER_BLK)
    cid3 = jnp.concatenate([cids, pad_i]).astype(f32).reshape(N_CBLK, 1,
                                                              COLS_PER_BLK)
    tgt = jnp.broadcast_to(targets.astype(f32)[:, None], (B, 8))
    cam = jnp.broadcast_to(cams.astype(f32)[:, None], (B, 8))

    sm, gm, acc = _run_pass1(inputs.astype(f32), proxy_p, tgt, cam, pid3, cid3)
    idx = _run_pass2(gm.swapaxes(0, 1).reshape(B, N_COARSE))
    table = sm.reshape(B * N_COARSE, ROWW)
    cand = _sc_gather(idx.reshape(B * IDX_W // 128, 128), table)
    loss = _run_pass3(cand.reshape(B, IDX_W * ROWW), acc)
    return loss[0, 0]


# speedup vs baseline: 7.8644x; 1.2080x over previous
import functools

import jax
import jax.numpy as jnp
from jax import lax
from jax.experimental import pallas as pl
from jax.experimental.pallas import tpu as pltpu
from jax.experimental.pallas import tpu_sc as plsc

NUM_FEATURES = 128
NUM_SAMPLES = 100000
N_PAD = 102400
NUM_HARDS = 50
TEMP = 0.07
B = 1024
ROWW = 128
N_COARSE = N_PAD // ROWW
COLS_PER_BLK = 2048
CRS_PER_BLK = COLS_PER_BLK // ROWW
N_CBLK = N_PAD // COLS_PER_BLK
ROWS1 = 256
ROWS2 = 256
ROWS3 = 256
IDX_W = 64
G_PER_TILE = B * IDX_W // 32
N_CHUNK = G_PER_TILE // 128
NEG_BIG = -9999999.0
NINF = -1e30


def _pass1_body(x_ref, p_ref, tgt_ref, cam_ref, pid_ref, cid_ref,
                sm_ref, gm_ref, acc_ref):
    j = pl.program_id(1)
    x = x_ref[...]
    nrm = jnp.sqrt(jnp.sum(x * x, axis=1, keepdims=True))
    xn = x / jnp.maximum(nrm, 1e-12)
    pb = p_ref[...]
    sims = lax.dot_general(xn, pb, (((1,), (1,)), ((), ())),
                           preferred_element_type=jnp.float32) / TEMP
    t = tgt_ref[:, 0:1]
    cm = cam_ref[:, 0:1]
    p = pid_ref[0]
    cd = cid_ref[0]
    neg = (t != p) & (p >= 0.0)
    pos = (t == p) & (cm != cd)
    sm = jnp.where(neg, sims, sims + NEG_BIG)
    sm3 = sm.reshape(ROWS1, CRS_PER_BLK, ROWW)
    sm_ref[...] = sm3
    gm_ref[...] = jnp.max(sm3, axis=2).reshape(1, ROWS1, CRS_PER_BLK)

    bm = jnp.max(jnp.where(pos, sims, NINF), axis=1, keepdims=True)
    e = jnp.where(pos, jnp.exp(sims - bm), 0.0)
    bsum = jnp.sum(e, axis=1, keepdims=True)
    bsp = jnp.sum(jnp.where(pos, sims, 0.0), axis=1, keepdims=True)
    bcnt = jnp.sum(jnp.where(pos, 1.0, 0.0), axis=1, keepdims=True)

    @pl.when(j == 0)
    def _():
        lanes = lax.broadcasted_iota(jnp.int32, (ROWS1, 8), 1)
        acc_ref[...] = jnp.where(lanes == 0, NINF, 0.0)

    mo = acc_ref[:, 0:1]
    so = acc_ref[:, 1:2]
    mn = jnp.maximum(mo, bm)
    sn = so * jnp.exp(mo - mn) + bsum * jnp.exp(bm - mn)
    acc_ref[:, 0:1] = mn
    acc_ref[:, 1:2] = sn
    acc_ref[:, 2:3] = acc_ref[:, 2:3] + bsp
    acc_ref[:, 3:4] = acc_ref[:, 3:4] + bcnt


def _pass2_body(gm_ref, idx_ref, v_ref):
    i0 = pl.program_id(0)
    v_ref[...] = gm_ref[...]
    colio = lax.broadcasted_iota(jnp.int32, (ROWS2, N_COARSE), 1)
    slot = lax.broadcasted_iota(jnp.int32, (ROWS2, IDX_W), 1)
    rbase = (lax.broadcasted_iota(jnp.int32, (ROWS2, 1), 0)
             + i0 * ROWS2) * N_COARSE

    def it(i, acc):
        v = v_ref[...]
        mi = jnp.max(v, axis=1, keepdims=True)
        am = jnp.max(jnp.where(v == mi, colio, -1), axis=1, keepdims=True)
        v_ref[...] = jnp.where(colio == am, -3.0e38, v)
        gidx = jnp.broadcast_to(rbase + am, (ROWS2, IDX_W))
        return jnp.where(slot == i, gidx, acc)

    acc = lax.fori_loop(0, NUM_HARDS,
                        it, jnp.zeros((ROWS2, IDX_W), jnp.int32))
    idx_ref[...] = acc


def _sc_gather_body(idx_hbm, table_hbm, out_hbm, idx_v, buf0, buf1,
                    sem0, sem1):
    c = lax.axis_index("c")
    s = lax.axis_index("s")
    wid = s * 2 + c
    pltpu.sync_copy(idx_hbm.at[pl.ds(wid * N_CHUNK, N_CHUNK)], idx_v)
    bufs = (buf0, buf1)
    sems = (sem0, sem1)
    nch = N_CHUNK // 2

    def fire(ch):
        b = ch % 2
        return [
            pltpu.async_copy(table_hbm.at[idx_v.at[2 * ch + r]],
                             bufs[b].at[pl.ds(r * 128, 128)], sems[b])
            for r in range(2)
        ]

    pend = fire(0)
    for ch in range(nch):
        nxt = fire(ch + 1) if ch + 1 < nch else []
        for cp in pend:
            cp.wait()
        pltpu.sync_copy(
            bufs[ch % 2],
            out_hbm.at[pl.ds(wid * G_PER_TILE + ch * 256, 256)])
        pend = nxt


def _pass3_body(cand_ref, acc_ref, out_ref, v_ref):
    i0 = pl.program_id(0)
    ncand = NUM_HARDS * ROWW
    cands = cand_ref[:, :ncand]
    v_ref[...] = cands
    m_top = jnp.max(cands, axis=1, keepdims=True)

    def it(i, stot):
        v = v_ref[...]
        mi = jnp.max(v, axis=1, keepdims=True)
        v_ref[...] = jnp.where(v == mi, -3.0e38, v)
        return stot + jnp.exp(mi - m_top)

    sneg = lax.fori_loop(0, NUM_HARDS, it, jnp.zeros((ROWS3, 1), jnp.float32))

    m_pos = acc_ref[:, 0:1]
    s_pos = acc_ref[:, 1:2]
    sum_pos = acc_ref[:, 2:3]
    cnt = acc_ref[:, 3:4]
    m = jnp.maximum(m_pos, m_top)
    lse = m + jnp.log(s_pos * jnp.exp(m_pos - m) + sneg * jnp.exp(m_top - m))
    mean_pos = sum_pos / jnp.maximum(cnt, 1.0)
    per_row = jnp.where(cnt > 0, lse - mean_pos, 0.0)
    partial = jnp.sum(per_row) * (1.0 / B)

    @pl.when(i0 == 0)
    def _():
        out_ref[...] = jnp.zeros((8, 128), jnp.float32)

    out_ref[...] = out_ref[...] + lax.broadcast_in_dim(partial, (8, 128), ())


def _run_pass1(xn, proxy_p, tgt, cam, pid3, cid3):
    return pl.pallas_call(
        _pass1_body,
        grid=(B // ROWS1, N_CBLK),
        in_specs=[
            pl.BlockSpec((ROWS1, NUM_FEATURES), lambda i, j: (i, 0)),
            pl.BlockSpec((COLS_PER_BLK, NUM_FEATURES), lambda i, j: (j, 0)),
            pl.BlockSpec((ROWS1, 8), lambda i, j: (i, 0)),
            pl.BlockSpec((ROWS1, 8), lambda i, j: (i, 0)),
            pl.BlockSpec((1, 1, COLS_PER_BLK), lambda i, j: (j, 0, 0)),
            pl.BlockSpec((1, 1, COLS_PER_BLK), lambda i, j: (j, 0, 0)),
        ],
        out_specs=[
            pl.BlockSpec((ROWS1, CRS_PER_BLK, ROWW), lambda i, j: (i, j, 0)),
            pl.BlockSpec((1, ROWS1, CRS_PER_BLK), lambda i, j: (j, i, 0)),
            pl.BlockSpec((ROWS1, 8), lambda i, j: (i, 0)),
        ],
        out_shape=[
            jax.ShapeDtypeStruct((B, N_COARSE, ROWW), jnp.float32),
            jax.ShapeDtypeStruct((N_CBLK, B, CRS_PER_BLK), jnp.float32),
            jax.ShapeDtypeStruct((B, 8), jnp.float32),
        ],
    )(xn, proxy_p, tgt, cam, pid3, cid3)


def _run_pass2(gm):
    return pl.pallas_call(
        _pass2_body,
        grid=(B // ROWS2,),
        in_specs=[pl.BlockSpec((ROWS2, N_COARSE), lambda i: (i, 0))],
        out_specs=pl.BlockSpec((ROWS2, IDX_W), lambda i: (i, 0)),
        out_shape=jax.ShapeDtypeStruct((B, IDX_W), jnp.int32),
        scratch_shapes=[pltpu.VMEM((ROWS2, N_COARSE), jnp.float32)],
    )(gm)


@functools.partial(
    pl.kernel,
    out_type=jax.ShapeDtypeStruct((B * IDX_W, ROWW), jnp.float32),
    mesh=plsc.VectorSubcoreMesh(core_axis_name="c", subcore_axis_name="s"),
    scratch_types=[
        pltpu.VMEM((N_CHUNK, 128), jnp.int32),
        pltpu.VMEM((256, ROWW), jnp.float32),
        pltpu.VMEM((256, ROWW), jnp.float32),
        pltpu.SemaphoreType.DMA,
        pltpu.SemaphoreType.DMA,
    ],
)
def _sc_gather(idx_hbm, table_hbm, out_hbm, idx_v, buf0, buf1, sem0, sem1):
    _sc_gather_body(idx_hbm, table_hbm, out_hbm, idx_v, buf0, buf1,
                    sem0, sem1)


def _run_pass3(cand, acc):
    return pl.pallas_call(
        _pass3_body,
        grid=(B // ROWS3,),
        in_specs=[
            pl.BlockSpec((ROWS3, IDX_W * ROWW), lambda i: (i, 0)),
            pl.BlockSpec((ROWS3, 8), lambda i: (i, 0)),
        ],
        out_specs=pl.BlockSpec((8, 128), lambda i: (0, 0)),
        out_shape=jax.ShapeDtypeStruct((8, 128), jnp.float32),
        scratch_shapes=[pltpu.VMEM((ROWS3, NUM_HARDS * ROWW), jnp.float32)],
    )(cand, acc)


def kernel(inputs, targets, cams, proxy, pids, cids):
    f32 = jnp.float32
    proxy_p = jnp.pad(proxy, ((0, N_PAD - NUM_SAMPLES), (0, 0)))
    pad_i = jnp.full((N_PAD - NUM_SAMPLES,), -1, pids.dtype)
    pid3 = jnp.concatenate([pids, pad_i]).astype(f32).reshape(N_CBLK, 1,
                                                              COLS_PER_BLK)
    cid3 = jnp.concatenate([cids, pad_i]).astype(f32).reshape(N_CBLK, 1,
                                                              COLS_PER_BLK)
    tgt = jnp.broadcast_to(targets.astype(f32)[:, None], (B, 8))
    cam = jnp.broadcast_to(cams.astype(f32)[:, None], (B, 8))

    sm, gm, acc = _run_pass1(inputs.astype(f32), proxy_p, tgt, cam, pid3, cid3)
    idx = _run_pass2(gm.swapaxes(0, 1).reshape(B, N_COARSE))
    table = sm.reshape(B * N_COARSE, ROWW)
    cand = _sc_gather(idx.reshape(B * IDX_W // 128, 128), table)
    loss = _run_pass3(cand.reshape(B, IDX_W * ROWW), acc)
    return loss[0, 0]


# speedup vs baseline: 9.5431x; 1.2135x over previous
import functools

import jax
import jax.numpy as jnp
from jax import lax
from jax.experimental import pallas as pl
from jax.experimental.pallas import tpu as pltpu
from jax.experimental.pallas import tpu_sc as plsc

NUM_FEATURES = 128
NUM_SAMPLES = 100000
N_PAD = 102400
NUM_HARDS = 50
TEMP = 0.07
B = 1024
ROWW = 128
N_COARSE = N_PAD // ROWW
COLS_PER_BLK = 2048
CRS_PER_BLK = COLS_PER_BLK // ROWW
N_CBLK = N_PAD // COLS_PER_BLK
CB = 512
ROWS1 = 256
ROWS2 = 256
ROWS3 = 256
IDX_W = 64
G_PER_TILE = CB * IDX_W // 32
N_CHUNK = G_PER_TILE // 128
NEG_BIG = -9999999.0
NINF = -1e30


def _pass1_body(x_ref, p_ref, tgt_ref, cam_ref, pid_ref, cid_ref,
                sm_ref, gm_ref, acc_ref):
    j = pl.program_id(1)
    x = x_ref[...]
    nrm = jnp.sqrt(jnp.sum(x * x, axis=1, keepdims=True))
    xn = x / jnp.maximum(nrm, 1e-12)
    pb = p_ref[...]
    sims = lax.dot_general(xn, pb, (((1,), (1,)), ((), ())),
                           preferred_element_type=jnp.float32) / TEMP
    t = tgt_ref[:, 0:1]
    cm = cam_ref[:, 0:1]
    p = pid_ref[0]
    cd = cid_ref[0]
    neg = (t != p) & (p >= 0.0)
    pos = (t == p) & (cm != cd)
    sm = jnp.where(neg, sims, sims + NEG_BIG)
    sm3 = sm.reshape(ROWS1, CRS_PER_BLK, ROWW)
    sm_ref[...] = sm3
    gm_ref[...] = jnp.max(sm3, axis=2).reshape(1, ROWS1, CRS_PER_BLK)

    bm = jnp.max(jnp.where(pos, sims, NINF), axis=1, keepdims=True)
    e = jnp.where(pos, jnp.exp(sims - bm), 0.0)
    bsum = jnp.sum(e, axis=1, keepdims=True)
    bsp = jnp.sum(jnp.where(pos, sims, 0.0), axis=1, keepdims=True)
    bcnt = jnp.sum(jnp.where(pos, 1.0, 0.0), axis=1, keepdims=True)

    @pl.when(j == 0)
    def _():
        lanes = lax.broadcasted_iota(jnp.int32, (ROWS1, 8), 1)
        acc_ref[...] = jnp.where(lanes == 0, NINF, 0.0)

    mo = acc_ref[:, 0:1]
    so = acc_ref[:, 1:2]
    mn = jnp.maximum(mo, bm)
    sn = so * jnp.exp(mo - mn) + bsum * jnp.exp(bm - mn)
    acc_ref[:, 0:1] = mn
    acc_ref[:, 1:2] = sn
    acc_ref[:, 2:3] = acc_ref[:, 2:3] + bsp
    acc_ref[:, 3:4] = acc_ref[:, 3:4] + bcnt


def _pass2_body(gm_ref, idx_ref, v_ref):
    i0 = pl.program_id(0)
    v_ref[...] = gm_ref[...]
    colio = lax.broadcasted_iota(jnp.int32, (ROWS2, N_COARSE), 1)
    slot = lax.broadcasted_iota(jnp.int32, (ROWS2, IDX_W), 1)
    rbase = (lax.broadcasted_iota(jnp.int32, (ROWS2, 1), 0)
             + i0 * ROWS2) * N_COARSE

    def it(i, acc):
        v = v_ref[...]
        mi = jnp.max(v, axis=1, keepdims=True)
        am = jnp.max(jnp.where(v == mi, colio, -1), axis=1, keepdims=True)
        v_ref[...] = jnp.where(colio == am, -3.0e38, v)
        gidx = jnp.broadcast_to(rbase + am, (ROWS2, IDX_W))
        return jnp.where(slot == i, gidx, acc)

    acc = lax.fori_loop(0, NUM_HARDS,
                        it, jnp.zeros((ROWS2, IDX_W), jnp.int32))
    idx_ref[...] = acc


def _sc_gather_body(idx_hbm, table_hbm, out_hbm, idx_v, buf0, buf1,
                    sem0, sem1):
    c = lax.axis_index("c")
    s = lax.axis_index("s")
    wid = s * 2 + c
    pltpu.sync_copy(idx_hbm.at[pl.ds(wid * N_CHUNK, N_CHUNK)], idx_v)
    bufs = (buf0, buf1)
    sems = (sem0, sem1)
    nch = N_CHUNK // 2

    def fire(ch):
        b = ch % 2
        return [
            pltpu.async_copy(table_hbm.at[idx_v.at[2 * ch + r]],
                             bufs[b].at[pl.ds(r * 128, 128)], sems[b])
            for r in range(2)
        ]

    pend = fire(0)
    for ch in range(nch):
        nxt = fire(ch + 1) if ch + 1 < nch else []
        for cp in pend:
            cp.wait()
        pltpu.sync_copy(
            bufs[ch % 2],
            out_hbm.at[pl.ds(wid * G_PER_TILE + ch * 256, 256)])
        pend = nxt


def _pass3_body(cand_ref, acc_ref, out_ref, v_ref):
    i0 = pl.program_id(0)
    ncand = NUM_HARDS * ROWW
    cands = cand_ref[:, :ncand]
    v_ref[...] = cands
    m_top = jnp.max(cands, axis=1, keepdims=True)

    def it(i, stot):
        v = v_ref[...]
        mi = jnp.max(v, axis=1, keepdims=True)
        v_ref[...] = jnp.where(v == mi, -3.0e38, v)
        return stot + jnp.exp(mi - m_top)

    sneg = lax.fori_loop(0, NUM_HARDS, it, jnp.zeros((ROWS3, 1), jnp.float32))

    m_pos = acc_ref[:, 0:1]
    s_pos = acc_ref[:, 1:2]
    sum_pos = acc_ref[:, 2:3]
    cnt = acc_ref[:, 3:4]
    m = jnp.maximum(m_pos, m_top)
    lse = m + jnp.log(s_pos * jnp.exp(m_pos - m) + sneg * jnp.exp(m_top - m))
    mean_pos = sum_pos / jnp.maximum(cnt, 1.0)
    per_row = jnp.where(cnt > 0, lse - mean_pos, 0.0)
    partial = jnp.sum(per_row) * (1.0 / B)

    @pl.when(i0 == 0)
    def _():
        out_ref[...] = jnp.zeros((8, 128), jnp.float32)

    out_ref[...] = out_ref[...] + lax.broadcast_in_dim(partial, (8, 128), ())


def _run_pass1(xn, proxy_p, tgt, cam, pid3, cid3):
    return pl.pallas_call(
        _pass1_body,
        grid=(CB // ROWS1, N_CBLK),
        in_specs=[
            pl.BlockSpec((ROWS1, NUM_FEATURES), lambda i, j: (i, 0)),
            pl.BlockSpec((COLS_PER_BLK, NUM_FEATURES), lambda i, j: (j, 0)),
            pl.BlockSpec((ROWS1, 8), lambda i, j: (i, 0)),
            pl.BlockSpec((ROWS1, 8), lambda i, j: (i, 0)),
            pl.BlockSpec((1, 1, COLS_PER_BLK), lambda i, j: (j, 0, 0)),
            pl.BlockSpec((1, 1, COLS_PER_BLK), lambda i, j: (j, 0, 0)),
        ],
        out_specs=[
            pl.BlockSpec((ROWS1, CRS_PER_BLK, ROWW), lambda i, j: (i, j, 0)),
            pl.BlockSpec((1, ROWS1, CRS_PER_BLK), lambda i, j: (j, i, 0)),
            pl.BlockSpec((ROWS1, 8), lambda i, j: (i, 0)),
        ],
        out_shape=[
            jax.ShapeDtypeStruct((CB, N_COARSE, ROWW), jnp.float32),
            jax.ShapeDtypeStruct((N_CBLK, CB, CRS_PER_BLK), jnp.float32),
            jax.ShapeDtypeStruct((CB, 8), jnp.float32),
        ],
    )(xn, proxy_p, tgt, cam, pid3, cid3)


def _run_pass2(gm):
    return pl.pallas_call(
        _pass2_body,
        grid=(CB // ROWS2,),
        in_specs=[pl.BlockSpec((ROWS2, N_COARSE), lambda i: (i, 0))],
        out_specs=pl.BlockSpec((ROWS2, IDX_W), lambda i: (i, 0)),
        out_shape=jax.ShapeDtypeStruct((CB, IDX_W), jnp.int32),
        scratch_shapes=[pltpu.VMEM((ROWS2, N_COARSE), jnp.float32)],
    )(gm)


@functools.partial(
    pl.kernel,
    out_type=jax.ShapeDtypeStruct((CB * IDX_W, ROWW), jnp.float32),
    mesh=plsc.VectorSubcoreMesh(core_axis_name="c", subcore_axis_name="s"),
    scratch_types=[
        pltpu.VMEM((N_CHUNK, 128), jnp.int32),
        pltpu.VMEM((256, ROWW), jnp.float32),
        pltpu.VMEM((256, ROWW), jnp.float32),
        pltpu.SemaphoreType.DMA,
        pltpu.SemaphoreType.DMA,
    ],
)
def _sc_gather(idx_hbm, table_hbm, out_hbm, idx_v, buf0, buf1, sem0, sem1):
    _sc_gather_body(idx_hbm, table_hbm, out_hbm, idx_v, buf0, buf1,
                    sem0, sem1)


def _run_pass3(cand, acc):
    return pl.pallas_call(
        _pass3_body,
        grid=(CB // ROWS3,),
        in_specs=[
            pl.BlockSpec((ROWS3, IDX_W * ROWW), lambda i: (i, 0)),
            pl.BlockSpec((ROWS3, 8), lambda i: (i, 0)),
        ],
        out_specs=pl.BlockSpec((8, 128), lambda i: (0, 0)),
        out_shape=jax.ShapeDtypeStruct((8, 128), jnp.float32),
        scratch_shapes=[pltpu.VMEM((ROWS3, NUM_HARDS * ROWW), jnp.float32)],
    )(cand, acc)


def kernel(inputs, targets, cams, proxy, pids, cids):
    f32 = jnp.float32
    proxy_p = jnp.pad(proxy, ((0, N_PAD - NUM_SAMPLES), (0, 0)))
    pad_i = jnp.full((N_PAD - NUM_SAMPLES,), -1, pids.dtype)
    pid3 = jnp.concatenate([pids, pad_i]).astype(f32).reshape(N_CBLK, 1,
                                                              COLS_PER_BLK)
    cid3 = jnp.concatenate([cids, pad_i]).astype(f32).reshape(N_CBLK, 1,
                                                              COLS_PER_BLK)
    tgt = jnp.broadcast_to(targets.astype(f32)[:, None], (B, 8))
    cam = jnp.broadcast_to(cams.astype(f32)[:, None], (B, 8))

    loss = jnp.float32(0.0)
    for c in range(B // CB):
        sl = slice(c * CB, (c + 1) * CB)
        sm, gm, acc = _run_pass1(inputs[sl].astype(f32), proxy_p,
                                 tgt[sl], cam[sl], pid3, cid3)
        idx = _run_pass2(gm.swapaxes(0, 1).reshape(CB, N_COARSE))
        table = sm.reshape(CB * N_COARSE, ROWW)
        cand = _sc_gather(idx.reshape(CB * IDX_W // 128, 128), table)
        out = _run_pass3(cand.reshape(CB, IDX_W * ROWW), acc)
        loss = loss + out[0, 0]
    return loss


# speedup vs baseline: 10.4495x; 1.0950x over previous
import functools

import jax
import jax.numpy as jnp
from jax import lax
from jax.experimental import pallas as pl
from jax.experimental.pallas import tpu as pltpu
from jax.experimental.pallas import tpu_sc as plsc

NUM_FEATURES = 128
NUM_SAMPLES = 100000
N_PAD = 102400
NUM_HARDS = 50
TEMP = 0.07
B = 1024
ROWW = 128
N_COARSE = N_PAD // ROWW
COLS_PER_BLK = 2048
CRS_PER_BLK = COLS_PER_BLK // ROWW
N_CBLK = N_PAD // COLS_PER_BLK
CB = 256
ROWS1 = 256
ROWS2 = 256
ROWS3 = 256
IDX_W = 64
G_PER_TILE = CB * IDX_W // 32
N_CHUNK = G_PER_TILE // 128
NEG_BIG = -9999999.0
NINF = -1e30


def _pass1_body(x_ref, p_ref, tgt_ref, cam_ref, pid_ref, cid_ref,
                sm_ref, gm_ref, acc_ref):
    j = pl.program_id(1)
    x = x_ref[...]
    nrm = jnp.sqrt(jnp.sum(x * x, axis=1, keepdims=True))
    xn = x / jnp.maximum(nrm, 1e-12)
    pb = p_ref[...]
    sims = lax.dot_general(xn, pb, (((1,), (1,)), ((), ())),
                           preferred_element_type=jnp.float32) / TEMP
    t = tgt_ref[:, 0:1]
    cm = cam_ref[:, 0:1]
    p = pid_ref[0]
    cd = cid_ref[0]
    neg = (t != p) & (p >= 0.0)
    pos = (t == p) & (cm != cd)
    sm = jnp.where(neg, sims, sims + NEG_BIG)
    sm3 = sm.reshape(ROWS1, CRS_PER_BLK, ROWW)
    sm_ref[...] = sm3
    gm_ref[...] = jnp.max(sm3, axis=2).reshape(1, ROWS1, CRS_PER_BLK)

    bm = jnp.max(jnp.where(pos, sims, NINF), axis=1, keepdims=True)
    e = jnp.where(pos, jnp.exp(sims - bm), 0.0)
    bsum = jnp.sum(e, axis=1, keepdims=True)
    bsp = jnp.sum(jnp.where(pos, sims, 0.0), axis=1, keepdims=True)
    bcnt = jnp.sum(jnp.where(pos, 1.0, 0.0), axis=1, keepdims=True)

    @pl.when(j == 0)
    def _():
        lanes = lax.broadcasted_iota(jnp.int32, (ROWS1, 8), 1)
        acc_ref[...] = jnp.where(lanes == 0, NINF, 0.0)

    mo = acc_ref[:, 0:1]
    so = acc_ref[:, 1:2]
    mn = jnp.maximum(mo, bm)
    sn = so * jnp.exp(mo - mn) + bsum * jnp.exp(bm - mn)
    acc_ref[:, 0:1] = mn
    acc_ref[:, 1:2] = sn
    acc_ref[:, 2:3] = acc_ref[:, 2:3] + bsp
    acc_ref[:, 3:4] = acc_ref[:, 3:4] + bcnt


def _pass2_body(gm_ref, idx_ref, v_ref):
    i0 = pl.program_id(0)
    v_ref[...] = gm_ref[...]
    colio = lax.broadcasted_iota(jnp.int32, (ROWS2, N_COARSE), 1)
    slot = lax.broadcasted_iota(jnp.int32, (ROWS2, IDX_W), 1)
    rbase = (lax.broadcasted_iota(jnp.int32, (ROWS2, 1), 0)
             + i0 * ROWS2) * N_COARSE

    def it(i, acc):
        v = v_ref[...]
        mi = jnp.max(v, axis=1, keepdims=True)
        am = jnp.max(jnp.where(v == mi, colio, -1), axis=1, keepdims=True)
        v_ref[...] = jnp.where(colio == am, -3.0e38, v)
        gidx = jnp.broadcast_to(rbase + am, (ROWS2, IDX_W))
        return jnp.where(slot == i, gidx, acc)

    acc = lax.fori_loop(0, NUM_HARDS,
                        it, jnp.zeros((ROWS2, IDX_W), jnp.int32))
    idx_ref[...] = acc


def _sc_gather_body(idx_hbm, table_hbm, out_hbm, idx_v, buf0, buf1,
                    sem0, sem1):
    c = lax.axis_index("c")
    s = lax.axis_index("s")
    wid = s * 2 + c
    pltpu.sync_copy(idx_hbm.at[wid], idx_v)
    bufs = (buf0, buf1)
    sems = (sem0, sem1)
    nch = N_CHUNK // 2

    def fire(ch):
        b = ch % 2
        return [
            pltpu.async_copy(table_hbm.at[idx_v.at[2 * ch + r]],
                             bufs[b].at[pl.ds(r * 128, 128)], sems[b])
            for r in range(2)
        ]

    pend = fire(0)
    for ch in range(nch):
        nxt = fire(ch + 1) if ch + 1 < nch else []
        for cp in pend:
            cp.wait()
        pltpu.sync_copy(
            bufs[ch % 2],
            out_hbm.at[pl.ds(wid * G_PER_TILE + ch * 256, 256)])
        pend = nxt


def _pass3_body(cand_ref, acc_ref, out_ref, v_ref):
    i0 = pl.program_id(0)
    ncand = NUM_HARDS * ROWW
    cands = cand_ref[:, :ncand]
    v_ref[...] = cands
    m_top = jnp.max(cands, axis=1, keepdims=True)

    def it(i, stot):
        v = v_ref[...]
        mi = jnp.max(v, axis=1, keepdims=True)
        v_ref[...] = jnp.where(v == mi, -3.0e38, v)
        return stot + jnp.exp(mi - m_top)

    sneg = lax.fori_loop(0, NUM_HARDS, it, jnp.zeros((ROWS3, 1), jnp.float32))

    m_pos = acc_ref[:, 0:1]
    s_pos = acc_ref[:, 1:2]
    sum_pos = acc_ref[:, 2:3]
    cnt = acc_ref[:, 3:4]
    m = jnp.maximum(m_pos, m_top)
    lse = m + jnp.log(s_pos * jnp.exp(m_pos - m) + sneg * jnp.exp(m_top - m))
    mean_pos = sum_pos / jnp.maximum(cnt, 1.0)
    per_row = jnp.where(cnt > 0, lse - mean_pos, 0.0)
    partial = jnp.sum(per_row) * (1.0 / B)

    @pl.when(i0 == 0)
    def _():
        out_ref[...] = jnp.zeros((8, 128), jnp.float32)

    out_ref[...] = out_ref[...] + lax.broadcast_in_dim(partial, (8, 128), ())


def _run_pass1(xn, proxy_p, tgt, cam, pid3, cid3):
    return pl.pallas_call(
        _pass1_body,
        grid=(CB // ROWS1, N_CBLK),
        in_specs=[
            pl.BlockSpec((ROWS1, NUM_FEATURES), lambda i, j: (i, 0)),
            pl.BlockSpec((COLS_PER_BLK, NUM_FEATURES), lambda i, j: (j, 0)),
            pl.BlockSpec((ROWS1, 8), lambda i, j: (i, 0)),
            pl.BlockSpec((ROWS1, 8), lambda i, j: (i, 0)),
            pl.BlockSpec((1, 1, COLS_PER_BLK), lambda i, j: (j, 0, 0)),
            pl.BlockSpec((1, 1, COLS_PER_BLK), lambda i, j: (j, 0, 0)),
        ],
        out_specs=[
            pl.BlockSpec((ROWS1, CRS_PER_BLK, ROWW), lambda i, j: (i, j, 0)),
            pl.BlockSpec((1, ROWS1, CRS_PER_BLK), lambda i, j: (j, i, 0)),
            pl.BlockSpec((ROWS1, 8), lambda i, j: (i, 0)),
        ],
        out_shape=[
            jax.ShapeDtypeStruct((CB, N_COARSE, ROWW), jnp.float32),
            jax.ShapeDtypeStruct((N_CBLK, CB, CRS_PER_BLK), jnp.float32),
            jax.ShapeDtypeStruct((CB, 8), jnp.float32),
        ],
    )(xn, proxy_p, tgt, cam, pid3, cid3)


def _run_pass2(gm):
    return pl.pallas_call(
        _pass2_body,
        grid=(CB // ROWS2,),
        in_specs=[pl.BlockSpec((ROWS2, N_COARSE), lambda i: (i, 0))],
        out_specs=pl.BlockSpec((ROWS2, IDX_W), lambda i: (i, 0)),
        out_shape=jax.ShapeDtypeStruct((CB, IDX_W), jnp.int32),
        scratch_shapes=[pltpu.VMEM((ROWS2, N_COARSE), jnp.float32)],
    )(gm)


@functools.partial(
    pl.kernel,
    out_type=jax.ShapeDtypeStruct((CB * IDX_W, ROWW), jnp.float32),
    mesh=plsc.VectorSubcoreMesh(core_axis_name="c", subcore_axis_name="s"),
    scratch_types=[
        pltpu.VMEM((N_CHUNK, 128), jnp.int32),
        pltpu.VMEM((256, ROWW), jnp.float32),
        pltpu.VMEM((256, ROWW), jnp.float32),
        pltpu.SemaphoreType.DMA,
        pltpu.SemaphoreType.DMA,
    ],
)
def _sc_gather(idx_hbm, table_hbm, out_hbm, idx_v, buf0, buf1, sem0, sem1):
    _sc_gather_body(idx_hbm, table_hbm, out_hbm, idx_v, buf0, buf1,
                    sem0, sem1)


def _run_pass3(cand, acc):
    return pl.pallas_call(
        _pass3_body,
        grid=(CB // ROWS3,),
        in_specs=[
            pl.BlockSpec((ROWS3, IDX_W * ROWW), lambda i: (i, 0)),
            pl.BlockSpec((ROWS3, 8), lambda i: (i, 0)),
        ],
        out_specs=pl.BlockSpec((8, 128), lambda i: (0, 0)),
        out_shape=jax.ShapeDtypeStruct((8, 128), jnp.float32),
        scratch_shapes=[pltpu.VMEM((ROWS3, NUM_HARDS * ROWW), jnp.float32)],
    )(cand, acc)


def kernel(inputs, targets, cams, proxy, pids, cids):
    f32 = jnp.float32
    proxy_p = jnp.pad(proxy, ((0, N_PAD - NUM_SAMPLES), (0, 0)))
    pad_i = jnp.full((N_PAD - NUM_SAMPLES,), -1, pids.dtype)
    pid3 = jnp.concatenate([pids, pad_i]).astype(f32).reshape(N_CBLK, 1,
                                                              COLS_PER_BLK)
    cid3 = jnp.concatenate([cids, pad_i]).astype(f32).reshape(N_CBLK, 1,
                                                              COLS_PER_BLK)
    tgt = jnp.broadcast_to(targets.astype(f32)[:, None], (B, 8))
    cam = jnp.broadcast_to(cams.astype(f32)[:, None], (B, 8))

    loss = jnp.float32(0.0)
    for c in range(B // CB):
        sl = slice(c * CB, (c + 1) * CB)
        sm, gm, acc = _run_pass1(inputs[sl].astype(f32), proxy_p,
                                 tgt[sl], cam[sl], pid3, cid3)
        idx = _run_pass2(gm.swapaxes(0, 1).reshape(CB, N_COARSE))
        table = sm.reshape(CB * N_COARSE, ROWW)
        cand = _sc_gather(idx.reshape(32, N_CHUNK, 128), table)
        out = _run_pass3(cand.reshape(CB, IDX_W * ROWW), acc)
        loss = loss + out[0, 0]
    return loss


# speedup vs baseline: 10.5832x; 1.0128x over previous
import functools

import jax
import jax.numpy as jnp
from jax import lax
from jax.experimental import pallas as pl
from jax.experimental.pallas import tpu as pltpu
from jax.experimental.pallas import tpu_sc as plsc

NUM_FEATURES = 128
NUM_SAMPLES = 100000
N_PAD = 102400
NUM_HARDS = 50
TEMP = 0.07
B = 1024
ROWW = 128
N_COARSE = N_PAD // ROWW
COLS_PER_BLK = 2048
CRS_PER_BLK = COLS_PER_BLK // ROWW
N_CBLK = N_PAD // COLS_PER_BLK
CB = 256
ROWS1 = 256
ROWS2 = 256
ROWS3 = 256
IDX_W = 64
G_PER_TILE = CB * IDX_W // 32
N_CHUNK = G_PER_TILE // 128
NEG_BIG = -9999999.0
NINF = -1e30


def _pass1_body(x_ref, p_ref, tgt_ref, cam_ref, pid_ref, cid_ref,
                sm_ref, gm_ref, acc_ref):
    j = pl.program_id(1)
    x = x_ref[...]
    nrm = jnp.sqrt(jnp.sum(x * x, axis=1, keepdims=True))
    xn = x / jnp.maximum(nrm, 1e-12)
    pb = p_ref[...]
    sims = lax.dot_general(xn, pb, (((1,), (1,)), ((), ())),
                           preferred_element_type=jnp.float32) / TEMP
    t = tgt_ref[:, 0:1]
    cm = cam_ref[:, 0:1]
    p = pid_ref[0]
    cd = cid_ref[0]
    eq = t == p
    neg = (p >= 0.0) & ~eq
    pos = eq & (cm != cd)
    sm = jnp.where(neg, sims, sims + NEG_BIG)
    sm3 = sm.reshape(ROWS1, CRS_PER_BLK, ROWW)
    sm_ref[...] = sm3
    gm_ref[...] = jnp.max(sm3, axis=2).reshape(1, ROWS1, CRS_PER_BLK)

    pe = jnp.where(pos, sims, NINF)
    bm = jnp.max(pe, axis=1, keepdims=True)
    bsum = jnp.sum(jnp.exp(pe - bm), axis=1, keepdims=True)
    bsp = jnp.sum(jnp.where(pos, sims, 0.0), axis=1, keepdims=True)
    bcnt = jnp.sum(jnp.where(pos, 1.0, 0.0), axis=1, keepdims=True)

    @pl.when(j == 0)
    def _():
        lanes = lax.broadcasted_iota(jnp.int32, (ROWS1, 8), 1)
        acc_ref[...] = jnp.where(lanes == 0, NINF, 0.0)

    mo = acc_ref[:, 0:1]
    so = acc_ref[:, 1:2]
    mn = jnp.maximum(mo, bm)
    sn = so * jnp.exp(mo - mn) + bsum * jnp.exp(bm - mn)
    acc_ref[:, 0:1] = mn
    acc_ref[:, 1:2] = sn
    acc_ref[:, 2:3] = acc_ref[:, 2:3] + bsp
    acc_ref[:, 3:4] = acc_ref[:, 3:4] + bcnt


def _pass2_body(gm_ref, idx_ref, v_ref):
    i0 = pl.program_id(0)
    v_ref[...] = gm_ref[...]
    colio = lax.broadcasted_iota(jnp.int32, (ROWS2, N_COARSE), 1)
    slot = lax.broadcasted_iota(jnp.int32, (ROWS2, IDX_W), 1)
    rbase = (lax.broadcasted_iota(jnp.int32, (ROWS2, 1), 0)
             + i0 * ROWS2) * N_COARSE

    def it(i, acc):
        v = v_ref[...]
        mi = jnp.max(v, axis=1, keepdims=True)
        am = jnp.max(jnp.where(v == mi, colio, -1), axis=1, keepdims=True)
        v_ref[...] = jnp.where(colio == am, -3.0e38, v)
        gidx = jnp.broadcast_to(rbase + am, (ROWS2, IDX_W))
        return jnp.where(slot == i, gidx, acc)

    acc = lax.fori_loop(0, NUM_HARDS,
                        it, jnp.zeros((ROWS2, IDX_W), jnp.int32))
    idx_ref[...] = acc


def _sc_gather_body(idx_hbm, table_hbm, out_hbm, idx_v, buf0, buf1,
                    sem0, sem1):
    c = lax.axis_index("c")
    s = lax.axis_index("s")
    wid = s * 2 + c
    pltpu.sync_copy(idx_hbm.at[wid], idx_v)
    bufs = (buf0, buf1)
    sems = (sem0, sem1)
    nch = N_CHUNK // 2

    def fire(ch):
        b = ch % 2
        return [
            pltpu.async_copy(table_hbm.at[idx_v.at[2 * ch + r]],
                             bufs[b].at[pl.ds(r * 128, 128)], sems[b])
            for r in range(2)
        ]

    pend = fire(0)
    for ch in range(nch):
        nxt = fire(ch + 1) if ch + 1 < nch else []
        for cp in pend:
            cp.wait()
        pltpu.sync_copy(
            bufs[ch % 2],
            out_hbm.at[pl.ds(wid * G_PER_TILE + ch * 256, 256)])
        pend = nxt


def _pass3_body(cand_ref, acc_ref, out_ref, v_ref):
    i0 = pl.program_id(0)
    ncand = NUM_HARDS * ROWW
    cands = cand_ref[:, :ncand]
    v_ref[...] = cands
    m_top = jnp.max(cands, axis=1, keepdims=True)

    def it(i, stot):
        v = v_ref[...]
        mi = jnp.max(v, axis=1, keepdims=True)
        v_ref[...] = jnp.where(v == mi, -3.0e38, v)
        return stot + jnp.exp(mi - m_top)

    sneg = lax.fori_loop(0, NUM_HARDS, it, jnp.zeros((ROWS3, 1), jnp.float32))

    m_pos = acc_ref[:, 0:1]
    s_pos = acc_ref[:, 1:2]
    sum_pos = acc_ref[:, 2:3]
    cnt = acc_ref[:, 3:4]
    m = jnp.maximum(m_pos, m_top)
    lse = m + jnp.log(s_pos * jnp.exp(m_pos - m) + sneg * jnp.exp(m_top - m))
    mean_pos = sum_pos / jnp.maximum(cnt, 1.0)
    per_row = jnp.where(cnt > 0, lse - mean_pos, 0.0)
    partial = jnp.sum(per_row) * (1.0 / B)

    @pl.when(i0 == 0)
    def _():
        out_ref[...] = jnp.zeros((8, 128), jnp.float32)

    out_ref[...] = out_ref[...] + lax.broadcast_in_dim(partial, (8, 128), ())


def _run_pass1(xn, proxy_p, tgt, cam, pid3, cid3):
    return pl.pallas_call(
        _pass1_body,
        grid=(CB // ROWS1, N_CBLK),
        in_specs=[
            pl.BlockSpec((ROWS1, NUM_FEATURES), lambda i, j: (i, 0)),
            pl.BlockSpec((COLS_PER_BLK, NUM_FEATURES), lambda i, j: (j, 0)),
            pl.BlockSpec((ROWS1, 8), lambda i, j: (i, 0)),
            pl.BlockSpec((ROWS1, 8), lambda i, j: (i, 0)),
            pl.BlockSpec((1, 1, COLS_PER_BLK), lambda i, j: (j, 0, 0)),
            pl.BlockSpec((1, 1, COLS_PER_BLK), lambda i, j: (j, 0, 0)),
        ],
        out_specs=[
            pl.BlockSpec((ROWS1, CRS_PER_BLK, ROWW), lambda i, j: (i, j, 0)),
            pl.BlockSpec((1, ROWS1, CRS_PER_BLK), lambda i, j: (j, i, 0)),
            pl.BlockSpec((ROWS1, 8), lambda i, j: (i, 0)),
        ],
        out_shape=[
            jax.ShapeDtypeStruct((CB, N_COARSE, ROWW), jnp.float32),
            jax.ShapeDtypeStruct((N_CBLK, CB, CRS_PER_BLK), jnp.float32),
            jax.ShapeDtypeStruct((CB, 8), jnp.float32),
        ],
    )(xn, proxy_p, tgt, cam, pid3, cid3)


def _run_pass2(gm):
    return pl.pallas_call(
        _pass2_body,
        grid=(CB // ROWS2,),
        in_specs=[pl.BlockSpec((ROWS2, N_COARSE), lambda i: (i, 0))],
        out_specs=pl.BlockSpec((ROWS2, IDX_W), lambda i: (i, 0)),
        out_shape=jax.ShapeDtypeStruct((CB, IDX_W), jnp.int32),
        scratch_shapes=[pltpu.VMEM((ROWS2, N_COARSE), jnp.float32)],
    )(gm)


@functools.partial(
    pl.kernel,
    out_type=jax.ShapeDtypeStruct((CB * IDX_W, ROWW), jnp.float32),
    mesh=plsc.VectorSubcoreMesh(core_axis_name="c", subcore_axis_name="s"),
    scratch_types=[
        pltpu.VMEM((N_CHUNK, 128), jnp.int32),
        pltpu.VMEM((256, ROWW), jnp.float32),
        pltpu.VMEM((256, ROWW), jnp.float32),
        pltpu.SemaphoreType.DMA,
        pltpu.SemaphoreType.DMA,
    ],
)
def _sc_gather(idx_hbm, table_hbm, out_hbm, idx_v, buf0, buf1, sem0, sem1):
    _sc_gather_body(idx_hbm, table_hbm, out_hbm, idx_v, buf0, buf1,
                    sem0, sem1)


def _run_pass3(cand, acc):
    return pl.pallas_call(
        _pass3_body,
        grid=(CB // ROWS3,),
        in_specs=[
            pl.BlockSpec((ROWS3, IDX_W * ROWW), lambda i: (i, 0)),
            pl.BlockSpec((ROWS3, 8), lambda i: (i, 0)),
        ],
        out_specs=pl.BlockSpec((8, 128), lambda i: (0, 0)),
        out_shape=jax.ShapeDtypeStruct((8, 128), jnp.float32),
        scratch_shapes=[pltpu.VMEM((ROWS3, NUM_HARDS * ROWW), jnp.float32)],
    )(cand, acc)


def kernel(inputs, targets, cams, proxy, pids, cids):
    f32 = jnp.float32
    proxy_p = jnp.pad(proxy, ((0, N_PAD - NUM_SAMPLES), (0, 0)))
    pad_i = jnp.full((N_PAD - NUM_SAMPLES,), -1, pids.dtype)
    pid3 = jnp.concatenate([pids, pad_i]).astype(f32).reshape(N_CBLK, 1,
                                                              COLS_PER_BLK)
    cid3 = jnp.concatenate([cids, pad_i]).astype(f32).reshape(N_CBLK, 1,
                                                              COLS_PER_BLK)
    tgt = jnp.broadcast_to(targets.astype(f32)[:, None], (B, 8))
    cam = jnp.broadcast_to(cams.astype(f32)[:, None], (B, 8))

    loss = jnp.float32(0.0)
    for c in range(B // CB):
        sl = slice(c * CB, (c + 1) * CB)
        sm, gm, acc = _run_pass1(inputs[sl].astype(f32), proxy_p,
                                 tgt[sl], cam[sl], pid3, cid3)
        idx = _run_pass2(gm.swapaxes(0, 1).reshape(CB, N_COARSE))
        table = sm.reshape(CB * N_COARSE, ROWW)
        cand = _sc_gather(idx.reshape(32, N_CHUNK, 128), table)
        out = _run_pass3(cand.reshape(CB, IDX_W * ROWW), acc)
        loss = loss + out[0, 0]
    return loss
